# baseline (speedup 1.0000x reference)
import functools

import jax
import jax.numpy as jnp
from jax import lax
from jax.experimental import pallas as pl
from jax.experimental.pallas import tpu as pltpu
from jax.experimental.pallas import tpu_sc as plsc


N = 10000
E = 320000
_ROW_TILE = 400

_NC = 2
_NS = 16
_B = 128
_BPT = 80
_EPAD = _NC * _NS * _BPT * _B
_ROWS_PER_TILE = N // _NS
_ZCH = 5


def _make_edge_kernel(hw, ow, ebody_fn):
    def body(s_tab, t_tab, h_tab, ew, srcv, dstv, part,
             sidxA, didxA, sidxB, didxB,
             srowsA, trowsA, hrowsA, ewrowsA, obufA,
             srowsB, trowsB, hrowsB, ewrowsB, obufB,
             acc, gsemA, gsemB):
        c = lax.axis_index("c")
        s = lax.axis_index("s")
        wid = s * _NC + c

        ii = lax.iota(jnp.int32, 16)
        consts = {
            "mask3": jnp.where(ii < 14, 1.0, 0.0),
            "zsel": jnp.zeros((16,), jnp.int32),
            "sel2": 1 + ii // 6,
            "sel3": jnp.minimum(1 + (16 + ii) // 6, 5),
        }

        def zbody(e, _):
            zv = jnp.zeros((16,), jnp.float32)
            for k in range(ow // 16):
                obufA[e, pl.ds(16 * k, 16)] = zv
            return _
        lax.fori_loop(0, _B, zbody, None)
        for k in range(_ZCH):
            pltpu.sync_copy(obufA.at[pl.ds(0, 125)],
                            acc.at[pl.ds(s * _ROWS_PER_TILE + k * 125, 125)])
        plsc.subcore_barrier()

        def fetch(b, sidx, didx, srows, trows, hrows, ewrows, gsem):
            base = (wid * _BPT + b) * _B
            pltpu.sync_copy(srcv.at[pl.ds(base, _B)], sidx)
            pltpu.sync_copy(dstv.at[pl.ds(base, _B)], didx)
            pltpu.async_copy(s_tab.at[sidx], srows, gsem)
            pltpu.async_copy(t_tab.at[didx], trows, gsem)
            pltpu.async_copy(h_tab.at[sidx], hrows, gsem)
            pltpu.async_copy(ew.at[pl.ds(base, _B)], ewrows, gsem)

        def phase(bnext, sidx, didx, srows, trows, hrows, ewrows, obuf,
                  gsem):
            pltpu.make_async_copy(s_tab.at[sidx], srows, gsem).wait()
            pltpu.make_async_copy(t_tab.at[didx], trows, gsem).wait()
            pltpu.make_async_copy(h_tab.at[sidx], hrows, gsem).wait()
            pltpu.make_async_copy(ew.at[pl.ds(0, _B)], ewrows, gsem).wait()

            def ebody(e, _):
                ebody_fn(e, srows, trows, ewrows, hrows, obuf, consts)
                return _
            lax.fori_loop(0, _B, ebody, None)
            pltpu.sync_copy(obuf, acc.at[didx], add=True)
            fetch(bnext, sidx, didx, srows, trows, hrows, ewrows, gsem)

        fetch(0, sidxA, didxA, srowsA, trowsA, hrowsA, ewrowsA, gsemA)
        fetch(1, sidxB, didxB, srowsB, trowsB, hrowsB, ewrowsB, gsemB)

        def outer(i, _):
            phase(jnp.minimum(2 * i + 2, _BPT - 1),
                  sidxA, didxA, srowsA, trowsA, hrowsA, ewrowsA, obufA,
                  gsemA)
            phase(jnp.minimum(2 * i + 3, _BPT - 1),
                  sidxB, didxB, srowsB, trowsB, hrowsB, ewrowsB, obufB,
                  gsemB)
            return _
        lax.fori_loop(0, _BPT // 2, outer, None)
        pltpu.make_async_copy(s_tab.at[sidxA], srowsA, gsemA).wait()
        pltpu.make_async_copy(t_tab.at[didxA], trowsA, gsemA).wait()
        pltpu.make_async_copy(h_tab.at[sidxA], hrowsA, gsemA).wait()
        pltpu.make_async_copy(ew.at[pl.ds(0, _B)], ewrowsA, gsemA).wait()
        pltpu.make_async_copy(s_tab.at[sidxB], srowsB, gsemB).wait()
        pltpu.make_async_copy(t_tab.at[didxB], trowsB, gsemB).wait()
        pltpu.make_async_copy(h_tab.at[sidxB], hrowsB, gsemB).wait()
        pltpu.make_async_copy(ew.at[pl.ds(0, _B)], ewrowsB, gsemB).wait()

        plsc.subcore_barrier()
        pltpu.sync_copy(acc.at[pl.ds(s * _ROWS_PER_TILE, _ROWS_PER_TILE)],
                        part.at[c, s])

    return body


def _ebody_p(e, srows, trows, ewrows, hrows, obuf, consts):
    lv = srows[e, :] + trows[e, :] + ewrows[e, :]
    lv = jnp.where(lv >= 0.0, lv, 0.2 * lv)
    exv = jnp.exp(lv)
    m1 = exv.at[consts["zsel"]].get(mode="promise_in_bounds")
    obuf[e, pl.ds(0, 16)] = exv
    obuf[e, pl.ds(16, 16)] = m1 * hrows[e, pl.ds(0, 16)]


def _ebody_1(e, srows, trows, ewrows, hrows, obuf, consts):
    lv = srows[e, :] + trows[e, :] + ewrows[e, :]
    lv = jnp.where(lv >= 0.0, lv, 0.2 * lv)
    exv = jnp.exp(lv)
    m2 = exv.at[consts["sel2"]].get(mode="promise_in_bounds")
    m3 = exv.at[consts["sel3"]].get(mode="promise_in_bounds") * consts["mask3"]
    obuf[e, pl.ds(0, 16)] = exv
    obuf[e, pl.ds(16, 16)] = m2 * hrows[e, pl.ds(0, 16)]
    obuf[e, pl.ds(32, 16)] = m3 * hrows[e, pl.ds(16, 16)]


def _edge_pass(body, hw, ow, s_tab, t_tab, h_tab, ew, srcv, dstv):
    mesh = plsc.VectorSubcoreMesh(core_axis_name="c", subcore_axis_name="s")
    return pl.kernel(
        body,
        out_type=jax.ShapeDtypeStruct((_NC, _NS, _ROWS_PER_TILE, ow),
                                      jnp.float32),
        mesh=mesh,
        scratch_types=[
            pltpu.VMEM((_B,), jnp.int32),
            pltpu.VMEM((_B,), jnp.int32),
            pltpu.VMEM((_B,), jnp.int32),
            pltpu.VMEM((_B,), jnp.int32),
            pltpu.VMEM((_B, 16), jnp.float32),
            pltpu.VMEM((_B, 16), jnp.float32),
            pltpu.VMEM((_B, hw), jnp.float32),
            pltpu.VMEM((_B, 16), jnp.float32),
            pltpu.VMEM((_B, ow), jnp.float32),
            pltpu.VMEM((_B, 16), jnp.float32),
            pltpu.VMEM((_B, 16), jnp.float32),
            pltpu.VMEM((_B, hw), jnp.float32),
            pltpu.VMEM((_B, 16), jnp.float32),
            pltpu.VMEM((_B, ow), jnp.float32),
            pltpu.VMEM_SHARED((N, ow), jnp.float32),
            pltpu.SemaphoreType.DMA,
            pltpu.SemaphoreType.DMA,
        ],
        compiler_params=pltpu.CompilerParams(use_tc_tiling_on_sc=False,
                                             needs_layout_passes=False),
    )(s_tab, t_tab, h_tab, ew, srcv, dstv)


_edge_body_p = _make_edge_kernel(16, 32, _ebody_p)
_edge_body_1 = _make_edge_kernel(32, 48, _ebody_1)


_EBLK = 1280
_NEB = E // _EBLK
_NEBP = _EPAD // _EBLK


def _ew_kernel(attr_ref, w_ref, out_ref):
    i = pl.program_id(0)
    ew = jax.lax.dot_general(
        attr_ref[...], w_ref[...], (((1,), (0,)), ((), ())),
        preferred_element_type=jnp.float32,
        precision=jax.lax.Precision.HIGHEST)
    row = i * _EBLK + jax.lax.broadcasted_iota(jnp.int32, (_EBLK, 16), 0)
    out_ref[...] = jnp.where(row < E, ew, -1e30)


def _ew_pass(edge_attr, wcat):
    return pl.pallas_call(
        _ew_kernel,
        grid=(_NEBP,),
        in_specs=[
            pl.BlockSpec((_EBLK, D_EDGE_PAD),
                         lambda i: (jnp.minimum(i, _NEB - 1), 0)),
            pl.BlockSpec((D_EDGE_PAD, 16), lambda i: (0, 0)),
        ],
        out_specs=pl.BlockSpec((_EBLK, 16), lambda i: (i, 0)),
        out_shape=jax.ShapeDtypeStruct((_EPAD, 16), jnp.float32),
    )(edge_attr, wcat)


D_EDGE_PAD = 4


def _adj_pass_kernel(adj_ref, s_ref, out_ref, ss_ref):
    i = pl.program_id(0)
    blk = adj_ref[...]
    out_ref[...] = jax.lax.dot_general(
        blk, s_ref[...], (((1,), (0,)), ((), ())),
        preferred_element_type=jnp.float32,
        precision=jax.lax.Precision.HIGHEST)
    part = jnp.sum(blk * blk)

    @pl.when(i == 0)
    def _init():
        ss_ref[...] = jnp.zeros_like(ss_ref[...])

    ss_ref[...] = ss_ref[...] + part


def _adj_pass(adj, s):
    out, ss = pl.pallas_call(
        _adj_pass_kernel,
        grid=(N // _ROW_TILE,),
        in_specs=[
            pl.BlockSpec((_ROW_TILE, N), lambda i: (i, 0)),
            pl.BlockSpec((N, 16), lambda i: (0, 0)),
        ],
        out_specs=[
            pl.BlockSpec((_ROW_TILE, 16), lambda i: (i, 0)),
            pl.BlockSpec((8, 128), lambda i: (0, 0)),
        ],
        out_shape=[
            jax.ShapeDtypeStruct((N, 16), jnp.float32),
            jax.ShapeDtypeStruct((8, 128), jnp.float32),
        ],
    )(adj, s)
    return out, ss[0, 0]


def _egat2_dense(x, adjw, W, a_src, a_dst, We):
    h = x @ W
    a_s = jnp.sum(h * a_src, axis=-1)
    a_d = jnp.sum(h * a_dst, axis=-1)
    logit = jax.nn.leaky_relu(a_s[:, None] + a_d[None, :] + adjw * We[0, 0], 0.2)
    m = jnp.max(logit, axis=0, keepdims=True)
    exl = jnp.exp(logit - m)
    alpha = exl / (jnp.sum(exl, axis=0, keepdims=True) + 1e-16)
    return alpha.T @ h


def kernel(x, edge_index, edge_attr, adj, W1p, a1ps, a1pd, We1p, W1, a1s, a1d, We1, W2p, a2ps, a2pd, We2p, W2, a2s, a2d, We2, gamma, beta):
    h_p = x @ W1p
    h1 = x @ W1
    h5 = h1.reshape(N, 5, 6)
    asp = jnp.sum(h_p * a1ps, axis=-1, keepdims=True)
    adp = jnp.sum(h_p * a1pd, axis=-1, keepdims=True)
    as1 = jnp.sum(h5 * a1s[None], axis=-1)
    ad1 = jnp.sum(h5 * a1d[None], axis=-1)
    zpad = jnp.zeros((N, 10), jnp.float32)
    s_tab = jnp.concatenate([asp, as1, zpad], axis=1)
    t_tab = jnp.concatenate([adp, ad1, zpad], axis=1)
    h1t = jnp.concatenate([h1, jnp.zeros((N, 2), jnp.float32)], axis=1)

    wcat = jnp.concatenate(
        [We1p, We1, jnp.zeros((4, 10), jnp.float32)], axis=1)
    ew = _ew_pass(edge_attr, wcat)
    srcv = jnp.concatenate(
        [edge_index[0], jnp.zeros((_EPAD - E,), jnp.int32)])
    dstv = jnp.concatenate(
        [edge_index[1], jnp.zeros((_EPAD - E,), jnp.int32)])

    pp = _edge_pass(_edge_body_p, 16, 32,
                    s_tab, t_tab, h_p, ew, srcv, dstv).reshape(_NC, N, 32)
    p1 = _edge_pass(_edge_body_1, 32, 48,
                    s_tab, t_tab, h1t, ew, srcv, dstv).reshape(_NC, N, 48)

    accp = pp[0] + pp[1]
    s_logits = accp[:, 16:32] / (accp[:, 0:1] + 1e-16)

    s = jax.nn.softmax(s_logits, axis=-1)
    adj_s, sum_adj2 = _adj_pass(adj, s)

    acc1 = p1[0] + p1[1]
    den1 = jnp.repeat(acc1[:, 1:6], 6, axis=1)
    x1 = acc1[:, 16:46] / (den1 + 1e-16)

    x1p = s.T @ x1
    adj1 = s.T @ adj_s
    sts = s.T @ s
    res2 = sum_adj2 - 2.0 * jnp.trace(adj1) + jnp.sum(sts * sts)
    link1 = jnp.sqrt(jnp.maximum(res2, 0.0)) / (N * N)
    ent1 = jnp.mean(jnp.sum(-s * jnp.log(s + 1e-15), axis=-1))
    reg1 = link1 + ent1

    s2_logits = _egat2_dense(x1p, adj1, W2p, a2ps, a2pd, We2p)
    x2 = _egat2_dense(x1p, adj1, W2, a2s, a2d, We2)
    s2 = jax.nn.softmax(s2_logits, axis=-1)
    x2p = s2.T @ x2
    link2 = jnp.linalg.norm(adj1 - s2 @ s2.T) / adj1.size
    ent2 = jnp.mean(jnp.sum(-s2 * jnp.log(s2 + 1e-15), axis=-1))
    reg2 = jnp.mean(link2) + ent2

    mu = jnp.mean(x2p, axis=0)
    var = jnp.var(x2p, axis=0)
    xn = (x2p - mu) / jnp.sqrt(var + 1e-5) * gamma + beta
    return xn, reg1 * 0.08 + reg2 * 0.1


# speedup vs baseline: 1.0250x; 1.0250x over previous
import functools

import jax
import jax.numpy as jnp
from jax import lax
from jax.experimental import pallas as pl
from jax.experimental.pallas import tpu as pltpu
from jax.experimental.pallas import tpu_sc as plsc


N = 10000
E = 320000
_ROW_TILE = 400

_NC = 2
_NS = 16
_B = 128
_BPT = 80
_EPAD = _NC * _NS * _BPT * _B
_ROWS_PER_TILE = N // _NS
_ZCH = 5


def _make_edge_kernel(hw, ow, ebody_fn):
    def body(s_tab, t_tab, h_tab, attr, wcat, srcv, dstv, part,
             sidxA, didxA, sidxB, didxB,
             srowsA, trowsA, hrowsA, arowsA, obufA,
             srowsB, trowsB, hrowsB, arowsB, obufB,
             wbuf, acc, gsemA, gsemB):
        c = lax.axis_index("c")
        s = lax.axis_index("s")
        wid = s * _NC + c

        pltpu.sync_copy(wcat, wbuf)
        ii = lax.iota(jnp.int32, 16)
        consts = {
            "mask3": jnp.where(ii < 14, 1.0, 0.0),
            "zsel": jnp.zeros((16,), jnp.int32),
            "sel2": 1 + ii // 6,
            "sel3": jnp.minimum(1 + (16 + ii) // 6, 5),
            "w": [wbuf[j, :] for j in range(4)],
            "cc": [jnp.full((16,), j, jnp.int32) for j in range(4)],
        }

        def zbody(e, _):
            zv = jnp.zeros((16,), jnp.float32)
            for k in range(ow // 16):
                obufA[e, pl.ds(16 * k, 16)] = zv
            return _
        lax.fori_loop(0, _B, zbody, None)
        for k in range(_ZCH):
            pltpu.sync_copy(obufA.at[pl.ds(0, 125)],
                            acc.at[pl.ds(s * _ROWS_PER_TILE + k * 125, 125)])

        @pl.when(s == 0)
        def _zero_sentinel():
            pltpu.sync_copy(obufA.at[pl.ds(0, 8)], acc.at[pl.ds(N, 8)])
        plsc.subcore_barrier()

        def fetch(b, sidx, didx, srows, trows, hrows, arows, gsem):
            base = (wid * _BPT + b) * _B
            pltpu.sync_copy(srcv.at[pl.ds(base, _B)], sidx)
            pltpu.sync_copy(dstv.at[pl.ds(base, _B)], didx)
            pltpu.async_copy(s_tab.at[sidx], srows, gsem)
            pltpu.async_copy(t_tab.at[didx], trows, gsem)
            pltpu.async_copy(h_tab.at[sidx], hrows, gsem)
            pltpu.async_copy(attr.at[pl.ds(base * 4, _B * 4)],
                             arows.at[pl.ds(0, _B * 4)], gsem)

        def waitg(sidx, didx, srows, trows, hrows, arows, gsem):
            pltpu.make_async_copy(s_tab.at[sidx], srows, gsem).wait()
            pltpu.make_async_copy(t_tab.at[didx], trows, gsem).wait()
            pltpu.make_async_copy(h_tab.at[sidx], hrows, gsem).wait()
            pltpu.make_async_copy(attr.at[pl.ds(0, _B * 4)],
                                  arows.at[pl.ds(0, _B * 4)], gsem).wait()

        def phase(bnext, sidx, didx, srows, trows, hrows, arows, obuf,
                  gsem):
            waitg(sidx, didx, srows, trows, hrows, arows, gsem)

            def ebody(i, _):
                ebody_fn(2 * i, srows, trows, arows, hrows, obuf, consts)
                ebody_fn(2 * i + 1, srows, trows, arows, hrows, obuf,
                         consts)
                return _
            lax.fori_loop(0, _B // 2, ebody, None)
            pltpu.sync_copy(obuf, acc.at[didx], add=True)
            fetch(bnext, sidx, didx, srows, trows, hrows, arows, gsem)

        fetch(0, sidxA, didxA, srowsA, trowsA, hrowsA, arowsA, gsemA)
        fetch(1, sidxB, didxB, srowsB, trowsB, hrowsB, arowsB, gsemB)

        def outer(i, _):
            phase(jnp.minimum(2 * i + 2, _BPT - 1),
                  sidxA, didxA, srowsA, trowsA, hrowsA, arowsA, obufA,
                  gsemA)
            phase(jnp.minimum(2 * i + 3, _BPT - 1),
                  sidxB, didxB, srowsB, trowsB, hrowsB, arowsB, obufB,
                  gsemB)
            return _
        lax.fori_loop(0, _BPT // 2, outer, None)
        waitg(sidxA, didxA, srowsA, trowsA, hrowsA, arowsA, gsemA)
        waitg(sidxB, didxB, srowsB, trowsB, hrowsB, arowsB, gsemB)

        plsc.subcore_barrier()
        pltpu.sync_copy(acc.at[pl.ds(s * _ROWS_PER_TILE, _ROWS_PER_TILE)],
                        part.at[c, s])

    return body


def _edge_logits(e, srows, trows, arows, consts):
    w = consts["w"]
    av = arows[pl.ds(4 * e, 16)]
    g = lambda k: av.at[consts["cc"][k]].get(mode="promise_in_bounds")
    lv = (srows[e, :] + trows[e, :]
          + g(0) * w[0] + g(1) * w[1] + g(2) * w[2] + g(3) * w[3])
    lv = jnp.where(lv >= 0.0, lv, 0.2 * lv)
    return jnp.exp(lv)


def _ebody_p(e, srows, trows, arows, hrows, obuf, consts):
    exv = _edge_logits(e, srows, trows, arows, consts)
    m1 = exv.at[consts["zsel"]].get(mode="promise_in_bounds")
    obuf[e, pl.ds(0, 16)] = exv
    obuf[e, pl.ds(16, 16)] = m1 * hrows[e, pl.ds(0, 16)]


def _ebody_1(e, srows, trows, arows, hrows, obuf, consts):
    exv = _edge_logits(e, srows, trows, arows, consts)
    m2 = exv.at[consts["sel2"]].get(mode="promise_in_bounds")
    m3 = exv.at[consts["sel3"]].get(mode="promise_in_bounds") * consts["mask3"]
    obuf[e, pl.ds(0, 16)] = exv
    obuf[e, pl.ds(16, 16)] = m2 * hrows[e, pl.ds(0, 16)]
    obuf[e, pl.ds(32, 16)] = m3 * hrows[e, pl.ds(16, 16)]


def _edge_pass(body, hw, ow, s_tab, t_tab, h_tab, attr, wcat, srcv, dstv):
    mesh = plsc.VectorSubcoreMesh(core_axis_name="c", subcore_axis_name="s")
    return pl.kernel(
        body,
        out_type=jax.ShapeDtypeStruct((_NC, _NS, _ROWS_PER_TILE, ow),
                                      jnp.float32),
        mesh=mesh,
        scratch_types=[
            pltpu.VMEM((_B,), jnp.int32),
            pltpu.VMEM((_B,), jnp.int32),
            pltpu.VMEM((_B,), jnp.int32),
            pltpu.VMEM((_B,), jnp.int32),
            pltpu.VMEM((_B, 16), jnp.float32),
            pltpu.VMEM((_B, 16), jnp.float32),
            pltpu.VMEM((_B, hw), jnp.float32),
            pltpu.VMEM((_B * 4 + 16,), jnp.float32),
            pltpu.VMEM((_B, ow), jnp.float32),
            pltpu.VMEM((_B, 16), jnp.float32),
            pltpu.VMEM((_B, 16), jnp.float32),
            pltpu.VMEM((_B, hw), jnp.float32),
            pltpu.VMEM((_B * 4 + 16,), jnp.float32),
            pltpu.VMEM((_B, ow), jnp.float32),
            pltpu.VMEM((4, 16), jnp.float32),
            pltpu.VMEM_SHARED((N + 8, ow), jnp.float32),
            pltpu.SemaphoreType.DMA,
            pltpu.SemaphoreType.DMA,
        ],
        compiler_params=pltpu.CompilerParams(use_tc_tiling_on_sc=False,
                                             needs_layout_passes=False),
    )(s_tab, t_tab, h_tab, attr, wcat, srcv, dstv)


_edge_body_p = _make_edge_kernel(16, 32, _ebody_p)
_edge_body_1 = _make_edge_kernel(32, 48, _ebody_1)


def _adj_pass_kernel(adj_ref, s_ref, out_ref, ss_ref):
    i = pl.program_id(0)
    blk = adj_ref[...]
    out_ref[...] = jax.lax.dot_general(
        blk, s_ref[...], (((1,), (0,)), ((), ())),
        preferred_element_type=jnp.float32,
        precision=jax.lax.Precision.HIGHEST)
    part = jnp.sum(blk * blk)

    @pl.when(i == 0)
    def _init():
        ss_ref[...] = jnp.zeros_like(ss_ref[...])

    ss_ref[...] = ss_ref[...] + part


def _adj_pass(adj, s):
    out, ss = pl.pallas_call(
        _adj_pass_kernel,
        grid=(N // _ROW_TILE,),
        in_specs=[
            pl.BlockSpec((_ROW_TILE, N), lambda i: (i, 0)),
            pl.BlockSpec((N, 16), lambda i: (0, 0)),
        ],
        out_specs=[
            pl.BlockSpec((_ROW_TILE, 16), lambda i: (i, 0)),
            pl.BlockSpec((8, 128), lambda i: (0, 0)),
        ],
        out_shape=[
            jax.ShapeDtypeStruct((N, 16), jnp.float32),
            jax.ShapeDtypeStruct((8, 128), jnp.float32),
        ],
    )(adj, s)
    return out, ss[0, 0]


def _egat2_dense(x, adjw, W, a_src, a_dst, We):
    h = x @ W
    a_s = jnp.sum(h * a_src, axis=-1)
    a_d = jnp.sum(h * a_dst, axis=-1)
    logit = jax.nn.leaky_relu(a_s[:, None] + a_d[None, :] + adjw * We[0, 0], 0.2)
    m = jnp.max(logit, axis=0, keepdims=True)
    exl = jnp.exp(logit - m)
    alpha = exl / (jnp.sum(exl, axis=0, keepdims=True) + 1e-16)
    return alpha.T @ h


def kernel(x, edge_index, edge_attr, adj, W1p, a1ps, a1pd, We1p, W1, a1s, a1d, We1, W2p, a2ps, a2pd, We2p, W2, a2s, a2d, We2, gamma, beta):
    h_p = x @ W1p
    h1 = x @ W1
    h5 = h1.reshape(N, 5, 6)
    asp = jnp.sum(h_p * a1ps, axis=-1, keepdims=True)
    adp = jnp.sum(h_p * a1pd, axis=-1, keepdims=True)
    as1 = jnp.sum(h5 * a1s[None], axis=-1)
    ad1 = jnp.sum(h5 * a1d[None], axis=-1)
    zpad = jnp.zeros((N, 10), jnp.float32)
    sent = jnp.full((8, 16), -1e30, jnp.float32)
    s_tab = jnp.concatenate(
        [jnp.concatenate([asp, as1, zpad], axis=1), sent])
    t_tab = jnp.concatenate(
        [jnp.concatenate([adp, ad1, zpad], axis=1), sent])
    hpt = jnp.concatenate([h_p, jnp.zeros((8, 16), jnp.float32)])
    h1t = jnp.concatenate(
        [h1, jnp.zeros((N, 2), jnp.float32)], axis=1)
    h1t = jnp.concatenate([h1t, jnp.zeros((8, 32), jnp.float32)])

    wcat = jnp.concatenate(
        [We1p, We1, jnp.zeros((4, 10), jnp.float32)], axis=1)
    attr = jnp.concatenate(
        [edge_attr, jnp.zeros((_EPAD - E, 4), jnp.float32)]).reshape(-1)
    srcv = jnp.concatenate(
        [edge_index[0], jnp.full((_EPAD - E,), N, jnp.int32)])
    dstv = jnp.concatenate(
        [edge_index[1], jnp.full((_EPAD - E,), N, jnp.int32)])

    pp = _edge_pass(_edge_body_p, 16, 32, s_tab, t_tab, hpt,
                    attr, wcat, srcv, dstv).reshape(_NC, N, 32)
    p1 = _edge_pass(_edge_body_1, 32, 48, s_tab, t_tab, h1t,
                    attr, wcat, srcv, dstv).reshape(_NC, N, 48)

    accp = pp[0] + pp[1]
    s_logits = accp[:, 16:32] / (accp[:, 0:1] + 1e-16)

    s = jax.nn.softmax(s_logits, axis=-1)
    adj_s, sum_adj2 = _adj_pass(adj, s)

    acc1 = p1[0] + p1[1]
    den1 = jnp.repeat(acc1[:, 1:6], 6, axis=1)
    x1 = acc1[:, 16:46] / (den1 + 1e-16)

    x1p = s.T @ x1
    adj1 = s.T @ adj_s
    sts = s.T @ s
    res2 = sum_adj2 - 2.0 * jnp.trace(adj1) + jnp.sum(sts * sts)
    link1 = jnp.sqrt(jnp.maximum(res2, 0.0)) / (N * N)
    ent1 = jnp.mean(jnp.sum(-s * jnp.log(s + 1e-15), axis=-1))
    reg1 = link1 + ent1

    s2_logits = _egat2_dense(x1p, adj1, W2p, a2ps, a2pd, We2p)
    x2 = _egat2_dense(x1p, adj1, W2, a2s, a2d, We2)
    s2 = jax.nn.softmax(s2_logits, axis=-1)
    x2p = s2.T @ x2
    link2 = jnp.linalg.norm(adj1 - s2 @ s2.T) / adj1.size
    ent2 = jnp.mean(jnp.sum(-s2 * jnp.log(s2 + 1e-15), axis=-1))
    reg2 = jnp.mean(link2) + ent2

    mu = jnp.mean(x2p, axis=0)
    var = jnp.var(x2p, axis=0)
    xn = (x2p - mu) / jnp.sqrt(var + 1e-5) * gamma + beta
    return xn, reg1 * 0.08 + reg2 * 0.1


# speedup vs baseline: 1.0954x; 1.0686x over previous
import functools

import jax
import jax.numpy as jnp
from jax import lax
from jax.experimental import pallas as pl
from jax.experimental.pallas import tpu as pltpu
from jax.experimental.pallas import tpu_sc as plsc


N = 10000
E = 320000
_ROW_TILE = 400

_NC = 2
_NS = 16
_B = 128
_BPT = 80
_EPAD = _NC * _NS * _BPT * _B
_ROWS_PER_TILE = N // _NS
_ZCH = 5


def _make_edge_kernel(hw, ow, ebody_fn):
    def body(s_tab, t_tab, h_tab, attr, wcat, srcv, dstv, part,
             sidxA, didxA, sidxB, didxB,
             srowsA, trowsA, hrowsA, arowsA, obufA,
             srowsB, trowsB, hrowsB, arowsB, obufB,
             wbuf, acc, gsemA, gsemB):
        c = lax.axis_index("c")
        s = lax.axis_index("s")
        wid = s * _NC + c

        pltpu.sync_copy(wcat, wbuf)
        ii = lax.iota(jnp.int32, 16)
        consts = {
            "mask3": jnp.where(ii < 14, 1.0, 0.0),
            "zsel": jnp.zeros((16,), jnp.int32),
            "sel2": 1 + ii // 6,
            "sel3": jnp.minimum(1 + (16 + ii) // 6, 5),
            "w": [wbuf[j, :] for j in range(4)],
            "cc": [jnp.full((16,), j, jnp.int32) for j in range(4)],
        }

        def zbody(e, _):
            zv = jnp.zeros((16,), jnp.float32)
            for k in range(ow // 16):
                obufA[e, pl.ds(16 * k, 16)] = zv
            return _
        lax.fori_loop(0, _B, zbody, None)
        for k in range(_ZCH):
            pltpu.sync_copy(obufA.at[pl.ds(0, 125)],
                            acc.at[pl.ds(s * _ROWS_PER_TILE + k * 125, 125)])

        @pl.when(s == 0)
        def _zero_sentinel():
            pltpu.sync_copy(obufA.at[pl.ds(0, 8)], acc.at[pl.ds(N, 8)])
        plsc.subcore_barrier()

        def fetch(b, sidx, didx, srows, trows, hrows, arows, gsem):
            base = (wid * _BPT + b) * _B
            pltpu.sync_copy(srcv.at[pl.ds(base, _B)], sidx)
            pltpu.sync_copy(dstv.at[pl.ds(base, _B)], didx)
            pltpu.async_copy(s_tab.at[sidx], srows, gsem)
            pltpu.async_copy(t_tab.at[didx], trows, gsem)
            pltpu.async_copy(h_tab.at[sidx], hrows, gsem)
            pltpu.async_copy(attr.at[pl.ds(base * 4, _B * 4)],
                             arows.at[pl.ds(0, _B * 4)], gsem)

        def waitg(sidx, didx, srows, trows, hrows, arows, gsem):
            pltpu.make_async_copy(s_tab.at[sidx], srows, gsem).wait()
            pltpu.make_async_copy(t_tab.at[didx], trows, gsem).wait()
            pltpu.make_async_copy(h_tab.at[sidx], hrows, gsem).wait()
            pltpu.make_async_copy(attr.at[pl.ds(0, _B * 4)],
                                  arows.at[pl.ds(0, _B * 4)], gsem).wait()

        def phase(bnext, sidx, didx, srows, trows, hrows, arows, obuf,
                  gsem):
            waitg(sidx, didx, srows, trows, hrows, arows, gsem)

            def ebody(i, _):
                ebody_fn(2 * i, srows, trows, arows, hrows, obuf, consts)
                ebody_fn(2 * i + 1, srows, trows, arows, hrows, obuf,
                         consts)
                return _
            lax.fori_loop(0, _B // 2, ebody, None)
            pltpu.sync_copy(obuf, acc.at[didx], add=True)
            fetch(bnext, sidx, didx, srows, trows, hrows, arows, gsem)

        fetch(0, sidxA, didxA, srowsA, trowsA, hrowsA, arowsA, gsemA)
        fetch(1, sidxB, didxB, srowsB, trowsB, hrowsB, arowsB, gsemB)

        def outer(i, _):
            phase(jnp.minimum(2 * i + 2, _BPT - 1),
                  sidxA, didxA, srowsA, trowsA, hrowsA, arowsA, obufA,
                  gsemA)
            phase(jnp.minimum(2 * i + 3, _BPT - 1),
                  sidxB, didxB, srowsB, trowsB, hrowsB, arowsB, obufB,
                  gsemB)
            return _
        lax.fori_loop(0, _BPT // 2, outer, None)
        waitg(sidxA, didxA, srowsA, trowsA, hrowsA, arowsA, gsemA)
        waitg(sidxB, didxB, srowsB, trowsB, hrowsB, arowsB, gsemB)

        plsc.subcore_barrier()
        pltpu.sync_copy(acc.at[pl.ds(s * _ROWS_PER_TILE, _ROWS_PER_TILE)],
                        part.at[c, s])

    return body


def _edge_logits(e, srows, trows, arows, consts):
    w = consts["w"]
    av = arows[pl.ds(4 * e, 16)]
    g = lambda k: av.at[consts["cc"][k]].get(mode="promise_in_bounds")
    lv = (srows[e, :] + trows[e, :]
          + g(0) * w[0] + g(1) * w[1] + g(2) * w[2] + g(3) * w[3])
    lv = jnp.where(lv >= 0.0, lv, 0.2 * lv)
    return jnp.exp(lv)


def _ebody_p(e, srows, trows, arows, hrows, obuf, consts):
    exv = _edge_logits(e, srows, trows, arows, consts)
    m1 = exv.at[consts["zsel"]].get(mode="promise_in_bounds")
    obuf[e, pl.ds(0, 16)] = exv
    obuf[e, pl.ds(16, 16)] = m1 * hrows[e, pl.ds(0, 16)]


def _ebody_1(e, srows, trows, arows, hrows, obuf, consts):
    exv = _edge_logits(e, srows, trows, arows, consts)
    m2 = exv.at[consts["sel2"]].get(mode="promise_in_bounds")
    m3 = exv.at[consts["sel3"]].get(mode="promise_in_bounds") * consts["mask3"]
    obuf[e, pl.ds(0, 16)] = exv
    obuf[e, pl.ds(16, 16)] = m2 * hrows[e, pl.ds(0, 16)]
    obuf[e, pl.ds(32, 16)] = m3 * hrows[e, pl.ds(16, 16)]


def _edge_pass(body, hw, ow, s_tab, t_tab, h_tab, attr, wcat, srcv, dstv):
    mesh = plsc.VectorSubcoreMesh(core_axis_name="c", subcore_axis_name="s")
    return pl.kernel(
        body,
        out_type=jax.ShapeDtypeStruct((_NC, _NS, _ROWS_PER_TILE, ow),
                                      jnp.float32),
        mesh=mesh,
        scratch_types=[
            pltpu.VMEM((_B,), jnp.int32),
            pltpu.VMEM((_B,), jnp.int32),
            pltpu.VMEM((_B,), jnp.int32),
            pltpu.VMEM((_B,), jnp.int32),
            pltpu.VMEM((_B, 16), jnp.float32),
            pltpu.VMEM((_B, 16), jnp.float32),
            pltpu.VMEM((_B, hw), jnp.float32),
            pltpu.VMEM((_B * 4 + 16,), jnp.float32),
            pltpu.VMEM((_B, ow), jnp.float32),
            pltpu.VMEM((_B, 16), jnp.float32),
            pltpu.VMEM((_B, 16), jnp.float32),
            pltpu.VMEM((_B, hw), jnp.float32),
            pltpu.VMEM((_B * 4 + 16,), jnp.float32),
            pltpu.VMEM((_B, ow), jnp.float32),
            pltpu.VMEM((4, 16), jnp.float32),
            pltpu.VMEM_SHARED((N + 8, ow), jnp.float32),
            pltpu.SemaphoreType.DMA,
            pltpu.SemaphoreType.DMA,
        ],
        compiler_params=pltpu.CompilerParams(use_tc_tiling_on_sc=False,
                                             needs_layout_passes=False),
    )(s_tab, t_tab, h_tab, attr, wcat, srcv, dstv)


_edge_body_p = _make_edge_kernel(16, 32, _ebody_p)
_edge_body_1 = _make_edge_kernel(32, 48, _ebody_1)


def _adj_pass_kernel(adj_ref, s_ref, out_ref, ss_ref):
    i = pl.program_id(0)
    blk = adj_ref[...]
    out_ref[...] = jax.lax.dot_general(
        blk, s_ref[...], (((1,), (0,)), ((), ())),
        preferred_element_type=jnp.float32,
        precision=jax.lax.Precision.HIGHEST)
    part = jnp.sum(blk * blk)

    @pl.when(i == 0)
    def _init():
        ss_ref[...] = jnp.zeros_like(ss_ref[...])

    ss_ref[...] = ss_ref[...] + part


def _adj_pass(adj, s):
    out, ss = pl.pallas_call(
        _adj_pass_kernel,
        grid=(N // _ROW_TILE,),
        in_specs=[
            pl.BlockSpec((_ROW_TILE, N), lambda i: (i, 0)),
            pl.BlockSpec((N, 16), lambda i: (0, 0)),
        ],
        out_specs=[
            pl.BlockSpec((_ROW_TILE, 16), lambda i: (i, 0)),
            pl.BlockSpec((8, 128), lambda i: (0, 0)),
        ],
        out_shape=[
            jax.ShapeDtypeStruct((N, 16), jnp.float32),
            jax.ShapeDtypeStruct((8, 128), jnp.float32),
        ],
    )(adj, s)
    return out, ss[0, 0]


def _egat2_dense(x, adjw, W, a_src, a_dst, We):
    h = x @ W
    a_s = jnp.sum(h * a_src, axis=-1)
    a_d = jnp.sum(h * a_dst, axis=-1)
    logit = jax.nn.leaky_relu(a_s[:, None] + a_d[None, :] + adjw * We[0, 0], 0.2)
    m = jnp.max(logit, axis=0, keepdims=True)
    exl = jnp.exp(logit - m)
    alpha = exl / (jnp.sum(exl, axis=0, keepdims=True) + 1e-16)
    return alpha.T @ h


def kernel(x, edge_index, edge_attr, adj, W1p, a1ps, a1pd, We1p, W1, a1s, a1d, We1, W2p, a2ps, a2pd, We2p, W2, a2s, a2d, We2, gamma, beta):
    h_p = x @ W1p
    h1 = x @ W1
    h5 = h1.reshape(N, 5, 6)
    asp = jnp.sum(h_p * a1ps, axis=-1, keepdims=True)
    adp = jnp.sum(h_p * a1pd, axis=-1, keepdims=True)
    as1 = jnp.sum(h5 * a1s[None], axis=-1)
    ad1 = jnp.sum(h5 * a1d[None], axis=-1)
    zpad = jnp.zeros((N, 10), jnp.float32)
    sent = jnp.full((8, 16), -1e30, jnp.float32)
    s_tab = jnp.concatenate(
        [jnp.concatenate([asp, as1, zpad], axis=1), sent])
    t_tab = jnp.concatenate(
        [jnp.concatenate([adp, ad1, zpad], axis=1), sent])
    hpt = jnp.concatenate([h_p, jnp.zeros((8, 16), jnp.float32)])
    h1t = jnp.concatenate(
        [h1, jnp.zeros((N, 2), jnp.float32)], axis=1)
    h1t = jnp.concatenate([h1t, jnp.zeros((8, 32), jnp.float32)])

    wcat = jnp.concatenate(
        [We1p, We1, jnp.zeros((4, 10), jnp.float32)], axis=1)
    attr = jnp.pad(edge_attr.reshape(-1), (0, (_EPAD - E) * 4))
    srcv = jnp.concatenate(
        [edge_index[0], jnp.full((_EPAD - E,), N, jnp.int32)])
    dstv = jnp.concatenate(
        [edge_index[1], jnp.full((_EPAD - E,), N, jnp.int32)])

    pp = _edge_pass(_edge_body_p, 16, 32, s_tab, t_tab, hpt,
                    attr, wcat, srcv, dstv).reshape(_NC, N, 32)
    accp = pp[0] + pp[1]
    s_logits = accp[:, 16:32] / (accp[:, 0:1] + 1e-16)
    s = jax.nn.softmax(s_logits, axis=-1)
    adj_s, sum_adj2 = _adj_pass(adj, s)

    p1 = _edge_pass(_edge_body_1, 32, 48, s_tab, t_tab, h1t,
                    attr, wcat, srcv, dstv).reshape(_NC, N, 48)
    acc1 = p1[0] + p1[1]
    den1 = jnp.repeat(acc1[:, 1:6], 6, axis=1)
    x1 = acc1[:, 16:46] / (den1 + 1e-16)

    x1p = s.T @ x1
    adj1 = s.T @ adj_s
    sts = s.T @ s
    res2 = sum_adj2 - 2.0 * jnp.trace(adj1) + jnp.sum(sts * sts)
    link1 = jnp.sqrt(jnp.maximum(res2, 0.0)) / (N * N)
    ent1 = jnp.mean(jnp.sum(-s * jnp.log(s + 1e-15), axis=-1))
    reg1 = link1 + ent1

    s2_logits = _egat2_dense(x1p, adj1, W2p, a2ps, a2pd, We2p)
    x2 = _egat2_dense(x1p, adj1, W2, a2s, a2d, We2)
    s2 = jax.nn.softmax(s2_logits, axis=-1)
    x2p = s2.T @ x2
    link2 = jnp.linalg.norm(adj1 - s2 @ s2.T) / adj1.size
    ent2 = jnp.mean(jnp.sum(-s2 * jnp.log(s2 + 1e-15), axis=-1))
    reg2 = jnp.mean(link2) + ent2

    mu = jnp.mean(x2p, axis=0)
    var = jnp.var(x2p, axis=0)
    xn = (x2p - mu) / jnp.sqrt(var + 1e-5) * gamma + beta
    return xn, reg1 * 0.08 + reg2 * 0.1


# speedup vs baseline: 1.3809x; 1.2606x over previous
import functools

import jax
import jax.numpy as jnp
from jax import lax
from jax.experimental import pallas as pl
from jax.experimental.pallas import tpu as pltpu
from jax.experimental.pallas import tpu_sc as plsc


N = 10000
E = 320000
_ROW_TILE = 400

_NC = 2
_NS = 16
_B = 128
_BPT = 80
_EPAD = _NC * _NS * _BPT * _B
_ROWS_PER_TILE = N // _NS
_ZCH = 5


def _make_edge_kernel(hw, ow, ebody_fn):
    def body(s_tab, t_tab, h_tab, attr, wcat, srcv, dstv, part,
             sidxA, didxA, sidxB, didxB,
             srowsA, trowsA, hrowsA, arowsA, obufA,
             srowsB, trowsB, hrowsB, arowsB, obufB,
             wbuf, acc, gsemA, gsemB):
        c = lax.axis_index("c")
        s = lax.axis_index("s")
        wid = s * _NC + c

        pltpu.sync_copy(wcat, wbuf)
        ii = lax.iota(jnp.int32, 16)
        consts = {
            "mask3": jnp.where(ii < 14, 1.0, 0.0),
            "zsel": jnp.zeros((16,), jnp.int32),
            "sel2": 1 + ii // 6,
            "sel3": jnp.minimum(1 + (16 + ii) // 6, 5),
            "w": [wbuf[j, :] for j in range(4)],
            "cc": [jnp.full((16,), j, jnp.int32) for j in range(4)],
        }

        def zbody(e, _):
            zv = jnp.zeros((16,), jnp.float32)
            for k in range(ow // 16):
                obufA[e, pl.ds(16 * k, 16)] = zv
            return _
        lax.fori_loop(0, _B, zbody, None)
        for k in range(_ZCH):
            pltpu.sync_copy(obufA.at[pl.ds(0, 125)],
                            acc.at[pl.ds(s * _ROWS_PER_TILE + k * 125, 125)])

        @pl.when(s == 0)
        def _zero_sentinel():
            pltpu.sync_copy(obufA.at[pl.ds(0, 8)], acc.at[pl.ds(N, 8)])
        plsc.subcore_barrier()

        def fetch(b, sidx, didx, srows, trows, hrows, arows, gsem):
            base = (wid * _BPT + b) * _B
            pltpu.sync_copy(srcv.at[pl.ds(base, _B)], sidx)
            pltpu.sync_copy(dstv.at[pl.ds(base, _B)], didx)
            pltpu.async_copy(s_tab.at[sidx], srows, gsem)
            pltpu.async_copy(t_tab.at[didx], trows, gsem)
            pltpu.async_copy(h_tab.at[sidx], hrows, gsem)
            pltpu.async_copy(attr.at[pl.ds(base * 4, _B * 4)],
                             arows.at[pl.ds(0, _B * 4)], gsem)

        def waitg(sidx, didx, srows, trows, hrows, arows, gsem):
            pltpu.make_async_copy(s_tab.at[sidx], srows, gsem).wait()
            pltpu.make_async_copy(t_tab.at[didx], trows, gsem).wait()
            pltpu.make_async_copy(h_tab.at[sidx], hrows, gsem).wait()
            pltpu.make_async_copy(attr.at[pl.ds(0, _B * 4)],
                                  arows.at[pl.ds(0, _B * 4)], gsem).wait()

        def phase(bnext, sidx, didx, srows, trows, hrows, arows, obuf,
                  gsem):
            waitg(sidx, didx, srows, trows, hrows, arows, gsem)

            def ebody(i, _):
                ebody_fn(2 * i, srows, trows, arows, hrows, obuf, consts)
                ebody_fn(2 * i + 1, srows, trows, arows, hrows, obuf,
                         consts)
                return _
            lax.fori_loop(0, _B // 2, ebody, None)
            pltpu.sync_copy(obuf, acc.at[didx], add=True)
            fetch(bnext, sidx, didx, srows, trows, hrows, arows, gsem)

        fetch(0, sidxA, didxA, srowsA, trowsA, hrowsA, arowsA, gsemA)
        fetch(1, sidxB, didxB, srowsB, trowsB, hrowsB, arowsB, gsemB)

        def outer(i, _):
            phase(jnp.minimum(2 * i + 2, _BPT - 1),
                  sidxA, didxA, srowsA, trowsA, hrowsA, arowsA, obufA,
                  gsemA)
            phase(jnp.minimum(2 * i + 3, _BPT - 1),
                  sidxB, didxB, srowsB, trowsB, hrowsB, arowsB, obufB,
                  gsemB)
            return _
        lax.fori_loop(0, _BPT // 2, outer, None)
        waitg(sidxA, didxA, srowsA, trowsA, hrowsA, arowsA, gsemA)
        waitg(sidxB, didxB, srowsB, trowsB, hrowsB, arowsB, gsemB)

        plsc.subcore_barrier()
        pltpu.sync_copy(acc.at[pl.ds(s * _ROWS_PER_TILE, _ROWS_PER_TILE)],
                        part.at[c, s])

    return body


def _edge_logits(e, srows, trows, arows, consts):
    w = consts["w"]
    av = arows[pl.ds(4 * e, 16)]
    g = lambda k: av.at[consts["cc"][k]].get(mode="promise_in_bounds")
    lv = (srows[e, :] + trows[e, :]
          + g(0) * w[0] + g(1) * w[1] + g(2) * w[2] + g(3) * w[3])
    lv = jnp.where(lv >= 0.0, lv, 0.2 * lv)
    return jnp.exp(lv)


def _ebody_p(e, srows, trows, arows, hrows, obuf, consts):
    exv = _edge_logits(e, srows, trows, arows, consts)
    m1 = exv.at[consts["zsel"]].get(mode="promise_in_bounds")
    obuf[e, pl.ds(0, 16)] = exv
    obuf[e, pl.ds(16, 16)] = m1 * hrows[e, pl.ds(0, 16)]


def _ebody_1(e, srows, trows, arows, hrows, obuf, consts):
    exv = _edge_logits(e, srows, trows, arows, consts)
    m2 = exv.at[consts["sel2"]].get(mode="promise_in_bounds")
    m3 = exv.at[consts["sel3"]].get(mode="promise_in_bounds") * consts["mask3"]
    obuf[e, pl.ds(0, 16)] = exv
    obuf[e, pl.ds(16, 16)] = m2 * hrows[e, pl.ds(0, 16)]
    obuf[e, pl.ds(32, 16)] = m3 * hrows[e, pl.ds(16, 16)]


def _ebody_c(e, srows, trows, arows, hrows, obuf, consts):
    exv = _edge_logits(e, srows, trows, arows, consts)
    m1 = exv.at[consts["zsel"]].get(mode="promise_in_bounds")
    m2 = exv.at[consts["sel2"]].get(mode="promise_in_bounds")
    m3 = exv.at[consts["sel3"]].get(mode="promise_in_bounds") * consts["mask3"]
    obuf[e, pl.ds(0, 16)] = exv
    obuf[e, pl.ds(16, 16)] = m1 * hrows[e, pl.ds(0, 16)]
    obuf[e, pl.ds(32, 16)] = m2 * hrows[e, pl.ds(16, 16)]
    obuf[e, pl.ds(48, 16)] = m3 * hrows[e, pl.ds(32, 16)]


def _edge_pass(body, hw, ow, s_tab, t_tab, h_tab, attr, wcat, srcv, dstv):
    mesh = plsc.VectorSubcoreMesh(core_axis_name="c", subcore_axis_name="s")
    return pl.kernel(
        body,
        out_type=jax.ShapeDtypeStruct((_NC, _NS, _ROWS_PER_TILE, ow),
                                      jnp.float32),
        mesh=mesh,
        scratch_types=[
            pltpu.VMEM((_B,), jnp.int32),
            pltpu.VMEM((_B,), jnp.int32),
            pltpu.VMEM((_B,), jnp.int32),
            pltpu.VMEM((_B,), jnp.int32),
            pltpu.VMEM((_B, 16), jnp.float32),
            pltpu.VMEM((_B, 16), jnp.float32),
            pltpu.VMEM((_B, hw), jnp.float32),
            pltpu.VMEM((_B * 4 + 16,), jnp.float32),
            pltpu.VMEM((_B, ow), jnp.float32),
            pltpu.VMEM((_B, 16), jnp.float32),
            pltpu.VMEM((_B, 16), jnp.float32),
            pltpu.VMEM((_B, hw), jnp.float32),
            pltpu.VMEM((_B * 4 + 16,), jnp.float32),
            pltpu.VMEM((_B, ow), jnp.float32),
            pltpu.VMEM((4, 16), jnp.float32),
            pltpu.VMEM_SHARED((N + 8, ow), jnp.float32),
            pltpu.SemaphoreType.DMA,
            pltpu.SemaphoreType.DMA,
        ],
        compiler_params=pltpu.CompilerParams(use_tc_tiling_on_sc=False,
                                             needs_layout_passes=False),
    )(s_tab, t_tab, h_tab, attr, wcat, srcv, dstv)


_edge_body_p = _make_edge_kernel(16, 32, _ebody_p)
_edge_body_1 = _make_edge_kernel(32, 48, _ebody_1)
_edge_body_c = _make_edge_kernel(48, 64, _ebody_c)


def _adj_pass_kernel(adj_ref, s_ref, out_ref, ss_ref):
    i = pl.program_id(0)
    blk = adj_ref[...]
    out_ref[...] = jax.lax.dot_general(
        blk, s_ref[...], (((1,), (0,)), ((), ())),
        preferred_element_type=jnp.float32,
        precision=jax.lax.Precision.HIGHEST)
    part = jnp.sum(blk * blk)

    @pl.when(i == 0)
    def _init():
        ss_ref[...] = jnp.zeros_like(ss_ref[...])

    ss_ref[...] = ss_ref[...] + part


def _adj_pass(adj, s):
    out, ss = pl.pallas_call(
        _adj_pass_kernel,
        grid=(N // _ROW_TILE,),
        in_specs=[
            pl.BlockSpec((_ROW_TILE, N), lambda i: (i, 0)),
            pl.BlockSpec((N, 16), lambda i: (0, 0)),
        ],
        out_specs=[
            pl.BlockSpec((_ROW_TILE, 16), lambda i: (i, 0)),
            pl.BlockSpec((8, 128), lambda i: (0, 0)),
        ],
        out_shape=[
            jax.ShapeDtypeStruct((N, 16), jnp.float32),
            jax.ShapeDtypeStruct((8, 128), jnp.float32),
        ],
    )(adj, s)
    return out, ss[0, 0]


def _egat2_dense(x, adjw, W, a_src, a_dst, We):
    h = x @ W
    a_s = jnp.sum(h * a_src, axis=-1)
    a_d = jnp.sum(h * a_dst, axis=-1)
    logit = jax.nn.leaky_relu(a_s[:, None] + a_d[None, :] + adjw * We[0, 0], 0.2)
    m = jnp.max(logit, axis=0, keepdims=True)
    exl = jnp.exp(logit - m)
    alpha = exl / (jnp.sum(exl, axis=0, keepdims=True) + 1e-16)
    return alpha.T @ h


def kernel(x, edge_index, edge_attr, adj, W1p, a1ps, a1pd, We1p, W1, a1s, a1d, We1, W2p, a2ps, a2pd, We2p, W2, a2s, a2d, We2, gamma, beta):
    h_p = x @ W1p
    h1 = x @ W1
    h5 = h1.reshape(N, 5, 6)
    asp = jnp.sum(h_p * a1ps, axis=-1, keepdims=True)
    adp = jnp.sum(h_p * a1pd, axis=-1, keepdims=True)
    as1 = jnp.sum(h5 * a1s[None], axis=-1)
    ad1 = jnp.sum(h5 * a1d[None], axis=-1)
    zpad = jnp.zeros((N, 10), jnp.float32)
    sent = jnp.full((8, 16), -1e30, jnp.float32)
    s_tab = jnp.concatenate(
        [jnp.concatenate([asp, as1, zpad], axis=1), sent])
    t_tab = jnp.concatenate(
        [jnp.concatenate([adp, ad1, zpad], axis=1), sent])
    hcat = jnp.concatenate(
        [h_p, h1, jnp.zeros((N, 2), jnp.float32)], axis=1)
    hcat = jnp.concatenate([hcat, jnp.zeros((8, 48), jnp.float32)])

    wcat = jnp.concatenate(
        [We1p, We1, jnp.zeros((4, 10), jnp.float32)], axis=1)
    attr = jnp.pad(edge_attr.reshape(-1), (0, (_EPAD - E) * 4))
    srcv = jnp.concatenate(
        [edge_index[0], jnp.full((_EPAD - E,), N, jnp.int32)])
    dstv = jnp.concatenate(
        [edge_index[1], jnp.full((_EPAD - E,), N, jnp.int32)])

    pc = _edge_pass(_edge_body_c, 48, 64, s_tab, t_tab, hcat,
                    attr, wcat, srcv, dstv).reshape(_NC, N, 64)
    acc = pc[0] + pc[1]
    s_logits = acc[:, 16:32] / (acc[:, 0:1] + 1e-16)
    s = jax.nn.softmax(s_logits, axis=-1)
    adj_s, sum_adj2 = _adj_pass(adj, s)

    den1 = jnp.repeat(acc[:, 1:6], 6, axis=1)
    x1 = acc[:, 32:62] / (den1 + 1e-16)

    x1p = s.T @ x1
    adj1 = s.T @ adj_s
    sts = s.T @ s
    res2 = sum_adj2 - 2.0 * jnp.trace(adj1) + jnp.sum(sts * sts)
    link1 = jnp.sqrt(jnp.maximum(res2, 0.0)) / (N * N)
    ent1 = jnp.mean(jnp.sum(-s * jnp.log(s + 1e-15), axis=-1))
    reg1 = link1 + ent1

    s2_logits = _egat2_dense(x1p, adj1, W2p, a2ps, a2pd, We2p)
    x2 = _egat2_dense(x1p, adj1, W2, a2s, a2d, We2)
    s2 = jax.nn.softmax(s2_logits, axis=-1)
    x2p = s2.T @ x2
    link2 = jnp.linalg.norm(adj1 - s2 @ s2.T) / adj1.size
    ent2 = jnp.mean(jnp.sum(-s2 * jnp.log(s2 + 1e-15), axis=-1))
    reg2 = jnp.mean(link2) + ent2

    mu = jnp.mean(x2p, axis=0)
    var = jnp.var(x2p, axis=0)
    xn = (x2p - mu) / jnp.sqrt(var + 1e-5) * gamma + beta
    return xn, reg1 * 0.08 + reg2 * 0.1
